# Initial kernel scaffold; baseline (speedup 1.0000x reference)
import jax
import jax.numpy as jnp
from jax.experimental import pallas as pl


def kernel(query, key, value, reference_points_cam, spatial_shapes, bev_mask, Wv, bv, Ws, bs_, Wa, ba, Wo, bo):
    raise NotImplementedError("write your pallas kernel here")


# speedup vs baseline: 1561.4622x; 1561.4622x over previous
import functools

import jax
import jax.numpy as jnp
from jax import lax
from jax.experimental import pallas as pl
from jax.experimental.pallas import tpu as pltpu
from jax.experimental.pallas import tpu_sc as plsc

B, N, C = 1, 2500, 128
S, M = 6, 4096
D = 4
H, W = 64, 64
HEADS, POINTS = 4, 8
HD = C // HEADS
NP = 2560
LANES = HEADS * POINTS
CPL = 4 * LANES

NC, NS = 2, 16
NW = NC * NS
UPW = (S * NP) // NW
SBU = 32
NSB = UPW // SBU


def _k1_body(v_ref, wv_ref, bv_ref, out_ref):
    out_ref[0, 0] = (
        jnp.dot(v_ref[...], wv_ref[0], preferred_element_type=jnp.float32, precision=lax.Precision.HIGHEST)
        + bv_ref[0, 0][None, :]
    )


def _value_table(vflat, Wv, bv):
    mb = 8
    wv_h = Wv.reshape(C, HEADS, HD).transpose(1, 0, 2)
    bv_h = bv.reshape(HEADS, 1, HD)
    return pl.pallas_call(
        _k1_body,
        grid=(S, mb, HEADS),
        in_specs=[
            pl.BlockSpec((512, C), lambda s, m, h: (s * mb + m, 0)),
            pl.BlockSpec((1, C, HD), lambda s, m, h: (h, 0, 0)),
            pl.BlockSpec((1, 1, HD), lambda s, m, h: (h, 0, 0)),
        ],
        out_specs=pl.BlockSpec((1, 1, 512, HD), lambda s, m, h: (s, h, m, 0)),
        out_shape=jax.ShapeDtypeStruct((S, HEADS, M, HD), jnp.float32),
    )(vflat, wv_h, bv_h)


def _iota2(shape, dim):
    return lax.broadcasted_iota(jnp.int32, shape, dim)


def _k2_body(q_ref, rp_ref, ws_ref, bs_ref, wa_ref, ba_ref, idx_ref, wgt_ref):
    s = pl.program_id(0)
    q = q_ref[...]
    off = jnp.dot(q, ws_ref[...], preferred_element_type=jnp.float32, precision=lax.Precision.HIGHEST) + bs_ref[0][None, :]
    a = jnp.dot(q, wa_ref[...], preferred_element_type=jnp.float32, precision=lax.Precision.HIGHEST) + ba_ref[0][None, :]

    a = a - jnp.max(a, axis=-1, keepdims=True)
    e = jnp.exp(a)
    seg = (_iota2((LANES, LANES), 0) // POINTS == _iota2((LANES, LANES), 1) // POINTS)
    attnw = e / jnp.dot(e, seg.astype(jnp.float32), preferred_element_type=jnp.float32, precision=lax.Precision.HIGHEST)

    j64 = _iota2((2 * LANES, LANES), 0)
    k64 = _iota2((2 * LANES, LANES), 1)
    ox = jnp.dot(off, (j64 == 2 * k64).astype(jnp.float32), preferred_element_type=jnp.float32, precision=lax.Precision.HIGHEST)
    oy = jnp.dot(off, (j64 == 2 * k64 + 1).astype(jnp.float32), preferred_element_type=jnp.float32, precision=lax.Precision.HIGHEST)

    rp8 = rp_ref[0]
    j8 = _iota2((2 * D, LANES), 0)
    d8 = (_iota2((2 * D, LANES), 1) % POINTS) % D
    rpx = jnp.dot(rp8, (j8 == 2 * d8).astype(jnp.float32), preferred_element_type=jnp.float32, precision=lax.Precision.HIGHEST)
    rpy = jnp.dot(rp8, (j8 == 2 * d8 + 1).astype(jnp.float32), preferred_element_type=jnp.float32, precision=lax.Precision.HIGHEST)

    x = rpx * W + ox - 0.5
    y = rpy * H + oy - 0.5
    x0 = jnp.floor(x)
    y0 = jnp.floor(y)
    fx = x - x0
    fy = y - y0

    hlane = _iota2((256, LANES), 1) // POINTS
    base = (s * HEADS + hlane) * M

    idxs = []
    wgts = []
    for cx, wx in ((x0, 1.0 - fx), (x0 + 1.0, fx)):
        for cy, wy in ((y0, 1.0 - fy), (y0 + 1.0, fy)):
            valid = ((cx >= 0.0) & (cx <= W - 1.0) & (cy >= 0.0) & (cy <= H - 1.0))
            wgts.append(wx * wy * attnw * valid.astype(jnp.float32))
            xi = jnp.clip(cx, 0.0, W - 1.0).astype(jnp.int32)
            yi = jnp.clip(cy, 0.0, H - 1.0).astype(jnp.int32)
            idxs.append(base + yi * W + xi)
    idx_ref[0] = jnp.concatenate(idxs, axis=-1)
    wgt_ref[0] = jnp.concatenate(wgts, axis=-1)


def _corner_data(qpad, rp8, Ws, bs_, Wa, ba):
    nb = NP // 256
    return pl.pallas_call(
        _k2_body,
        grid=(S, nb),
        in_specs=[
            pl.BlockSpec((256, C), lambda s, n: (n, 0)),
            pl.BlockSpec((1, 256, 2 * D), lambda s, n: (s, n, 0)),
            pl.BlockSpec((C, 2 * LANES), lambda s, n: (0, 0)),
            pl.BlockSpec((1, 2 * LANES), lambda s, n: (0, 0)),
            pl.BlockSpec((C, LANES), lambda s, n: (0, 0)),
            pl.BlockSpec((1, LANES), lambda s, n: (0, 0)),
        ],
        out_specs=[
            pl.BlockSpec((1, 256, CPL), lambda s, n: (s, n, 0)),
            pl.BlockSpec((1, 256, CPL), lambda s, n: (s, n, 0)),
        ],
        out_shape=[
            jax.ShapeDtypeStruct((S, NP, CPL), jnp.int32),
            jax.ShapeDtypeStruct((S, NP, CPL), jnp.float32),
        ],
    )(qpad, rp8, Ws, bs_.reshape(1, 2 * LANES), Wa, ba.reshape(1, LANES))


def _sc_sample(table, idxf, wgtf):
    mesh = plsc.VectorSubcoreMesh(core_axis_name="c", subcore_axis_name="s")

    @functools.partial(
        pl.kernel,
        mesh=mesh,
        compiler_params=pltpu.CompilerParams(use_tc_tiling_on_sc=False),
        out_type=jax.ShapeDtypeStruct((S * NP * HEADS, HD), jnp.float32),
        scratch_types=[
            pltpu.VMEM((SBU * CPL,), jnp.int32),
            pltpu.VMEM((SBU * CPL,), jnp.float32),
            pltpu.VMEM((CPL, HD), jnp.float32),
            pltpu.VMEM((SBU * HEADS, HD), jnp.float32),
            pltpu.SemaphoreType.DMA,
        ],
    )
    def run(table_hbm, idx_hbm, wgt_hbm, o_hbm, idxb, wgtb, rows, outb, sem):
        wid = lax.axis_index("s") * NC + lax.axis_index("c")
        base_u = wid * UPW

        def sb_body(sb, carry):
            u0 = base_u + sb * SBU
            pltpu.sync_copy(idx_hbm.at[pl.ds(u0 * CPL, SBU * CPL)], idxb)
            pltpu.sync_copy(wgt_hbm.at[pl.ds(u0 * CPL, SBU * CPL)], wgtb)

            def n_body(i, c2):
                pltpu.async_copy(
                    table_hbm.at[idxb.at[pl.ds(i * CPL, CPL)]], rows, sem
                ).wait()
                accs = [[jnp.zeros((16,), jnp.float32),
                         jnp.zeros((16,), jnp.float32)] for _ in range(HEADS)]
                for c in range(4):
                    wv0 = wgtb[pl.ds(i * CPL + c * LANES, 16)]
                    wv1 = wgtb[pl.ds(i * CPL + c * LANES + 16, 16)]
                    for h in range(HEADS):
                        wv = wv0 if h < 2 else wv1
                        for p in range(POINTS):
                            w = wv[(h % 2) * POINTS + p]
                            r = c * LANES + h * POINTS + p
                            accs[h][0] = accs[h][0] + rows[r, pl.ds(0, 16)] * w
                            accs[h][1] = accs[h][1] + rows[r, pl.ds(16, 16)] * w
                for h in range(HEADS):
                    outb[i * HEADS + h, pl.ds(0, 16)] = accs[h][0]
                    outb[i * HEADS + h, pl.ds(16, 16)] = accs[h][1]
                return c2

            lax.fori_loop(0, SBU, n_body, 0)
            pltpu.sync_copy(outb, o_hbm.at[pl.ds(u0 * HEADS, SBU * HEADS)])
            return carry

        lax.fori_loop(0, NSB, sb_body, 0)

    return run(table, idxf, wgtf)


def _k3_body(o_ref, m_ref, q_ref, wo_ref, bo_ref, out_ref):
    act = (jnp.sum(m_ref[...], axis=2) > 0.0).astype(jnp.float32)
    o = o_ref[...]
    acc = jnp.sum(o * act[:, :, None], axis=0)
    cnt = jnp.maximum(jnp.sum(act, axis=0), 1.0)
    slots = acc / cnt[:, None]
    out_ref[...] = (
        jnp.dot(slots, wo_ref[...], preferred_element_type=jnp.float32, precision=lax.Precision.HIGHEST)
        + bo_ref[0][None, :]
        + q_ref[...]
    )


def _finish(o, maskf, qpad, Wo, bo):
    nb = NP // 256
    return pl.pallas_call(
        _k3_body,
        grid=(nb,),
        in_specs=[
            pl.BlockSpec((S, 256, C), lambda n: (0, n, 0)),
            pl.BlockSpec((S, 256, D), lambda n: (0, n, 0)),
            pl.BlockSpec((256, C), lambda n: (n, 0)),
            pl.BlockSpec((C, C), lambda n: (0, 0)),
            pl.BlockSpec((1, C), lambda n: (0, 0)),
        ],
        out_specs=pl.BlockSpec((256, C), lambda n: (n, 0)),
        out_shape=jax.ShapeDtypeStruct((NP, C), jnp.float32),
    )(o, maskf, qpad, Wo, bo.reshape(1, C))


def kernel(query, key, value, reference_points_cam, spatial_shapes, bev_mask,
           Wv, bv, Ws, bs_, Wa, ba, Wo, bo):
    del key, spatial_shapes
    vflat = value[:, :, 0, :].reshape(S * M, C)
    qpad = jnp.pad(query[0], ((0, NP - N), (0, 0)))
    rp8 = jnp.pad(reference_points_cam[:, 0].reshape(S, N, 2 * D),
                  ((0, 0), (0, NP - N), (0, 0)))
    maskf = jnp.pad(bev_mask[:, 0].astype(jnp.float32),
                    ((0, 0), (0, NP - N), (0, 0)))

    table = _value_table(vflat, Wv, bv).reshape(S * HEADS * M, HD)
    idx, wgt = _corner_data(qpad, rp8, Ws, bs_, Wa, ba)
    o = _sc_sample(table, idx.reshape(S * NP * CPL), wgt.reshape(S * NP * CPL))
    o = o.reshape(S, NP, C)
    out = _finish(o, maskf, qpad, Wo, bo)
    return out[None, :N, :]


# speedup vs baseline: 1699.9110x; 1.0887x over previous
import functools

import jax
import jax.numpy as jnp
from jax import lax
from jax.experimental import pallas as pl
from jax.experimental.pallas import tpu as pltpu
from jax.experimental.pallas import tpu_sc as plsc

B, N, C = 1, 2500, 128
S, M = 6, 4096
D = 4
H, W = 64, 64
HEADS, POINTS = 4, 8
HD = C // HEADS
NP = 2560
LANES = HEADS * POINTS
CPL = 4 * LANES

NC, NS = 2, 16
NW = NC * NS
UPW = (S * NP) // NW
SBU = 32
NSB = UPW // SBU


def _k1_body(v_ref, wv_ref, bv_ref, out_ref):
    out_ref[0, 0] = (
        jnp.dot(v_ref[...], wv_ref[0], preferred_element_type=jnp.float32, precision=lax.Precision.HIGHEST)
        + bv_ref[0, 0][None, :]
    )


def _value_table(vflat, Wv, bv):
    mb = 8
    wv_h = Wv.reshape(C, HEADS, HD).transpose(1, 0, 2)
    bv_h = bv.reshape(HEADS, 1, HD)
    return pl.pallas_call(
        _k1_body,
        grid=(S, mb, HEADS),
        in_specs=[
            pl.BlockSpec((512, C), lambda s, m, h: (s * mb + m, 0)),
            pl.BlockSpec((1, C, HD), lambda s, m, h: (h, 0, 0)),
            pl.BlockSpec((1, 1, HD), lambda s, m, h: (h, 0, 0)),
        ],
        out_specs=pl.BlockSpec((1, 1, 512, HD), lambda s, m, h: (s, h, m, 0)),
        out_shape=jax.ShapeDtypeStruct((S, HEADS, M, HD), jnp.float32),
    )(vflat, wv_h, bv_h)


def _iota2(shape, dim):
    return lax.broadcasted_iota(jnp.int32, shape, dim)


def _k2_body(q_ref, rp_ref, ws_ref, bs_ref, wa_ref, ba_ref, idx_ref, wgt_ref):
    s = pl.program_id(0)
    q = q_ref[...]
    off = jnp.dot(q, ws_ref[...], preferred_element_type=jnp.float32, precision=lax.Precision.HIGHEST) + bs_ref[0][None, :]
    a = jnp.dot(q, wa_ref[...], preferred_element_type=jnp.float32, precision=lax.Precision.HIGHEST) + ba_ref[0][None, :]

    a = a - jnp.max(a, axis=-1, keepdims=True)
    e = jnp.exp(a)
    seg = (_iota2((LANES, LANES), 0) // POINTS == _iota2((LANES, LANES), 1) // POINTS)
    attnw = e / jnp.dot(e, seg.astype(jnp.float32), preferred_element_type=jnp.float32, precision=lax.Precision.HIGHEST)

    j64 = _iota2((2 * LANES, LANES), 0)
    k64 = _iota2((2 * LANES, LANES), 1)
    ox = jnp.dot(off, (j64 == 2 * k64).astype(jnp.float32), preferred_element_type=jnp.float32, precision=lax.Precision.HIGHEST)
    oy = jnp.dot(off, (j64 == 2 * k64 + 1).astype(jnp.float32), preferred_element_type=jnp.float32, precision=lax.Precision.HIGHEST)

    rp8 = rp_ref[0]
    j8 = _iota2((2 * D, LANES), 0)
    d8 = (_iota2((2 * D, LANES), 1) % POINTS) % D
    rpx = jnp.dot(rp8, (j8 == 2 * d8).astype(jnp.float32), preferred_element_type=jnp.float32, precision=lax.Precision.HIGHEST)
    rpy = jnp.dot(rp8, (j8 == 2 * d8 + 1).astype(jnp.float32), preferred_element_type=jnp.float32, precision=lax.Precision.HIGHEST)

    x = rpx * W + ox - 0.5
    y = rpy * H + oy - 0.5
    x0 = jnp.floor(x)
    y0 = jnp.floor(y)
    fx = x - x0
    fy = y - y0

    hlane = _iota2((256, LANES), 1) // POINTS
    base = (s * HEADS + hlane) * M

    idxs = []
    wgts = []
    for cx, wx in ((x0, 1.0 - fx), (x0 + 1.0, fx)):
        for cy, wy in ((y0, 1.0 - fy), (y0 + 1.0, fy)):
            valid = ((cx >= 0.0) & (cx <= W - 1.0) & (cy >= 0.0) & (cy <= H - 1.0))
            wgts.append(wx * wy * attnw * valid.astype(jnp.float32))
            xi = jnp.clip(cx, 0.0, W - 1.0).astype(jnp.int32)
            yi = jnp.clip(cy, 0.0, H - 1.0).astype(jnp.int32)
            idxs.append(base + yi * W + xi)
    idx_ref[0] = jnp.concatenate(idxs, axis=-1)
    wgt_ref[0] = jnp.concatenate(wgts, axis=-1)


def _corner_data(qpad, rp8, Ws, bs_, Wa, ba):
    nb = NP // 256
    return pl.pallas_call(
        _k2_body,
        grid=(S, nb),
        in_specs=[
            pl.BlockSpec((256, C), lambda s, n: (n, 0)),
            pl.BlockSpec((1, 256, 2 * D), lambda s, n: (s, n, 0)),
            pl.BlockSpec((C, 2 * LANES), lambda s, n: (0, 0)),
            pl.BlockSpec((1, 2 * LANES), lambda s, n: (0, 0)),
            pl.BlockSpec((C, LANES), lambda s, n: (0, 0)),
            pl.BlockSpec((1, LANES), lambda s, n: (0, 0)),
        ],
        out_specs=[
            pl.BlockSpec((1, 256, CPL), lambda s, n: (s, n, 0)),
            pl.BlockSpec((1, 256, CPL), lambda s, n: (s, n, 0)),
        ],
        out_shape=[
            jax.ShapeDtypeStruct((S, NP, CPL), jnp.int32),
            jax.ShapeDtypeStruct((S, NP, CPL), jnp.float32),
        ],
    )(qpad, rp8, Ws, bs_.reshape(1, 2 * LANES), Wa, ba.reshape(1, LANES))


def _sc_sample(table, idxf, wgtf):
    mesh = plsc.VectorSubcoreMesh(core_axis_name="c", subcore_axis_name="s")

    @functools.partial(
        pl.kernel,
        mesh=mesh,
        compiler_params=pltpu.CompilerParams(use_tc_tiling_on_sc=False),
        out_type=jax.ShapeDtypeStruct((S * NP * HEADS, HD), jnp.float32),
        scratch_types=[
            pltpu.VMEM((SBU * CPL,), jnp.int32),
            pltpu.VMEM((SBU * CPL,), jnp.float32),
            pltpu.VMEM((CPL, HD), jnp.float32),
            pltpu.VMEM((CPL, HD), jnp.float32),
            pltpu.VMEM((SBU * HEADS, HD), jnp.float32),
            pltpu.SemaphoreType.DMA,
            pltpu.SemaphoreType.DMA,
        ],
    )
    def run(table_hbm, idx_hbm, wgt_hbm, o_hbm, idxb, wgtb, rows0, rows1,
            outb, sem0, sem1):
        wid = lax.axis_index("s") * NC + lax.axis_index("c")
        base_u = wid * UPW

        def start(i, rbuf, sem_):
            pltpu.make_async_copy(
                table_hbm.at[idxb.at[pl.ds(i * CPL, CPL)]], rbuf, sem_
            ).start()

        def wait(rbuf, sem_):
            pltpu.make_async_copy(
                table_hbm.at[idxb.at[pl.ds(0, CPL)]], rbuf, sem_
            ).wait()

        def compute(i, rows):
            accs = [[jnp.zeros((16,), jnp.float32),
                     jnp.zeros((16,), jnp.float32)] for _ in range(HEADS)]
            for c in range(4):
                wv0 = wgtb[pl.ds(i * CPL + c * LANES, 16)]
                wv1 = wgtb[pl.ds(i * CPL + c * LANES + 16, 16)]
                for h in range(HEADS):
                    wv = wv0 if h < 2 else wv1
                    for p in range(POINTS):
                        w = wv[(h % 2) * POINTS + p]
                        r = c * LANES + h * POINTS + p
                        accs[h][0] = accs[h][0] + rows[r, pl.ds(0, 16)] * w
                        accs[h][1] = accs[h][1] + rows[r, pl.ds(16, 16)] * w
            for h in range(HEADS):
                outb[i * HEADS + h, pl.ds(0, 16)] = accs[h][0]
                outb[i * HEADS + h, pl.ds(16, 16)] = accs[h][1]

        def sb_body(sb, carry):
            u0 = base_u + sb * SBU
            pltpu.sync_copy(idx_hbm.at[pl.ds(u0 * CPL, SBU * CPL)], idxb)
            pltpu.sync_copy(wgt_hbm.at[pl.ds(u0 * CPL, SBU * CPL)], wgtb)
            start(0, rows0, sem0)

            def pair_body(j, c2):
                wait(rows0, sem0)
                start(2 * j + 1, rows1, sem1)
                compute(2 * j, rows0)
                wait(rows1, sem1)

                @pl.when(j < SBU // 2 - 1)
                def _():
                    start(2 * j + 2, rows0, sem0)

                compute(2 * j + 1, rows1)
                return c2

            lax.fori_loop(0, SBU // 2, pair_body, 0)
            pltpu.sync_copy(outb, o_hbm.at[pl.ds(u0 * HEADS, SBU * HEADS)])
            return carry

        lax.fori_loop(0, NSB, sb_body, 0)

    return run(table, idxf, wgtf)


def _k3_body(o_ref, m_ref, q_ref, wo_ref, bo_ref, out_ref):
    act = (jnp.sum(m_ref[...], axis=2) > 0.0).astype(jnp.float32)
    o = o_ref[...]
    acc = jnp.sum(o * act[:, :, None], axis=0)
    cnt = jnp.maximum(jnp.sum(act, axis=0), 1.0)
    slots = acc / cnt[:, None]
    out_ref[...] = (
        jnp.dot(slots, wo_ref[...], preferred_element_type=jnp.float32, precision=lax.Precision.HIGHEST)
        + bo_ref[0][None, :]
        + q_ref[...]
    )


def _finish(o, maskf, qpad, Wo, bo):
    nb = NP // 256
    return pl.pallas_call(
        _k3_body,
        grid=(nb,),
        in_specs=[
            pl.BlockSpec((S, 256, C), lambda n: (0, n, 0)),
            pl.BlockSpec((S, 256, D), lambda n: (0, n, 0)),
            pl.BlockSpec((256, C), lambda n: (n, 0)),
            pl.BlockSpec((C, C), lambda n: (0, 0)),
            pl.BlockSpec((1, C), lambda n: (0, 0)),
        ],
        out_specs=pl.BlockSpec((256, C), lambda n: (n, 0)),
        out_shape=jax.ShapeDtypeStruct((NP, C), jnp.float32),
    )(o, maskf, qpad, Wo, bo.reshape(1, C))


def kernel(query, key, value, reference_points_cam, spatial_shapes, bev_mask,
           Wv, bv, Ws, bs_, Wa, ba, Wo, bo):
    del key, spatial_shapes
    vflat = value[:, :, 0, :].reshape(S * M, C)
    qpad = jnp.pad(query[0], ((0, NP - N), (0, 0)))
    rp8 = jnp.pad(reference_points_cam[:, 0].reshape(S, N, 2 * D),
                  ((0, 0), (0, NP - N), (0, 0)))
    maskf = jnp.pad(bev_mask[:, 0].astype(jnp.float32),
                    ((0, 0), (0, NP - N), (0, 0)))

    table = _value_table(vflat, Wv, bv).reshape(S * HEADS * M, HD)
    idx, wgt = _corner_data(qpad, rp8, Ws, bs_, Wa, ba)
    o = _sc_sample(table, idx.reshape(S * NP * CPL), wgt.reshape(S * NP * CPL))
    o = o.reshape(S, NP, C)
    out = _finish(o, maskf, qpad, Wo, bo)
    return out[None, :N, :]


# speedup vs baseline: 1709.2437x; 1.0055x over previous
import functools

import jax
import jax.numpy as jnp
from jax import lax
from jax.experimental import pallas as pl
from jax.experimental.pallas import tpu as pltpu
from jax.experimental.pallas import tpu_sc as plsc

B, N, C = 1, 2500, 128
S, M = 6, 4096
D = 4
H, W = 64, 64
HEADS, POINTS = 4, 8
HD = C // HEADS
NP = 2560
LANES = HEADS * POINTS
CPL = 4 * LANES

NC, NS = 2, 16
NW = NC * NS
UPW = (S * NP) // NW
SBU = 32
NSB = UPW // SBU


def _k1_body(v_ref, wv_ref, bv_ref, out_ref):
    out_ref[...] = (
        jnp.dot(v_ref[...], wv_ref[0], preferred_element_type=jnp.float32)
        + bv_ref[0, 0][None, :]
    )


def _value_table(vflat4, Wv, bv):
    mb = 8
    wv_r = Wv.reshape(C, HEADS, HD)
    wbig = (jnp.eye(4, dtype=jnp.float32)[:, :, None, None, None]
            * wv_r[None, None])
    wbig = wbig.transpose(3, 0, 2, 1, 4).reshape(HEADS, 4 * C, C)
    bvt = jnp.tile(bv.reshape(HEADS, 1, HD), (1, 1, 4))
    return pl.pallas_call(
        _k1_body,
        grid=(S, mb, HEADS),
        in_specs=[
            pl.BlockSpec((128, 4 * C), lambda s, m, h: (s * mb + m, 0)),
            pl.BlockSpec((1, 4 * C, C), lambda s, m, h: (h, 0, 0)),
            pl.BlockSpec((1, 1, C), lambda s, m, h: (h, 0, 0)),
        ],
        out_specs=pl.BlockSpec((128, C), lambda s, m, h: ((s * HEADS + h) * mb + m, 0)),
        out_shape=jax.ShapeDtypeStruct((S * HEADS * M // 4, C), jnp.float32),
    )(vflat4, wbig, bvt)


def _iota2(shape, dim):
    return lax.broadcasted_iota(jnp.int32, shape, dim)


def _k2_body(q_ref, rp_ref, ws_ref, bs_ref, wa_ref, ba_ref, idx_ref, wgt_ref):
    q = q_ref[...]
    off = jnp.dot(q, ws_ref[...], preferred_element_type=jnp.float32, precision=lax.Precision.HIGHEST) + bs_ref[0][None, :]
    a = jnp.dot(q, wa_ref[...], preferred_element_type=jnp.float32, precision=lax.Precision.HIGHEST) + ba_ref[0][None, :]

    a = a - jnp.max(a, axis=-1, keepdims=True)
    e = jnp.exp(a)
    seg = (_iota2((LANES, LANES), 0) // POINTS == _iota2((LANES, LANES), 1) // POINTS)
    attnw = e / jnp.dot(e, seg.astype(jnp.float32), preferred_element_type=jnp.float32, precision=lax.Precision.HIGHEST)

    j64 = _iota2((2 * LANES, LANES), 0)
    k64 = _iota2((2 * LANES, LANES), 1)
    ox = jnp.dot(off, (j64 == 2 * k64).astype(jnp.float32), preferred_element_type=jnp.float32, precision=lax.Precision.HIGHEST)
    oy = jnp.dot(off, (j64 == 2 * k64 + 1).astype(jnp.float32), preferred_element_type=jnp.float32, precision=lax.Precision.HIGHEST)

    j8 = _iota2((2 * D, LANES), 0)
    d8 = (_iota2((2 * D, LANES), 1) % POINTS) % D
    selx = (j8 == 2 * d8).astype(jnp.float32)
    sely = (j8 == 2 * d8 + 1).astype(jnp.float32)
    hlane = _iota2((256, LANES), 1) // POINTS

    for s in range(S):
        rp8 = rp_ref[s]
        rpx = jnp.dot(rp8, selx, preferred_element_type=jnp.float32, precision=lax.Precision.HIGHEST)
        rpy = jnp.dot(rp8, sely, preferred_element_type=jnp.float32, precision=lax.Precision.HIGHEST)

        x = rpx * W + ox - 0.5
        y = rpy * H + oy - 0.5
        x0 = jnp.floor(x)
        y0 = jnp.floor(y)
        fx = x - x0
        fy = y - y0

        base = (s * HEADS + hlane) * M

        idxs = []
        wgts = []
        for cx, wx in ((x0, 1.0 - fx), (x0 + 1.0, fx)):
            for cy, wy in ((y0, 1.0 - fy), (y0 + 1.0, fy)):
                valid = ((cx >= 0.0) & (cx <= W - 1.0) & (cy >= 0.0) & (cy <= H - 1.0))
                wgts.append(wx * wy * attnw * valid.astype(jnp.float32))
                xi = jnp.clip(cx, 0.0, W - 1.0).astype(jnp.int32)
                yi = jnp.clip(cy, 0.0, H - 1.0).astype(jnp.int32)
                idxs.append(base + yi * W + xi)
        idx_ref[s] = jnp.concatenate(idxs, axis=-1)
        wgt_ref[s] = jnp.concatenate(wgts, axis=-1)


def _corner_data(qpad, rp8, Ws, bs_, Wa, ba):
    nb = NP // 256
    return pl.pallas_call(
        _k2_body,
        grid=(nb,),
        in_specs=[
            pl.BlockSpec((256, C), lambda n: (n, 0)),
            pl.BlockSpec((S, 256, 2 * D), lambda n: (0, n, 0)),
            pl.BlockSpec((C, 2 * LANES), lambda n: (0, 0)),
            pl.BlockSpec((1, 2 * LANES), lambda n: (0, 0)),
            pl.BlockSpec((C, LANES), lambda n: (0, 0)),
            pl.BlockSpec((1, LANES), lambda n: (0, 0)),
        ],
        out_specs=[
            pl.BlockSpec((S, 256, CPL), lambda n: (0, n, 0)),
            pl.BlockSpec((S, 256, CPL), lambda n: (0, n, 0)),
        ],
        out_shape=[
            jax.ShapeDtypeStruct((S, NP, CPL), jnp.int32),
            jax.ShapeDtypeStruct((S, NP, CPL), jnp.float32),
        ],
    )(qpad, rp8, Ws, bs_.reshape(1, 2 * LANES), Wa, ba.reshape(1, LANES))


def _sc_sample(table, idxf, wgtf):
    mesh = plsc.VectorSubcoreMesh(core_axis_name="c", subcore_axis_name="s")

    @functools.partial(
        pl.kernel,
        mesh=mesh,
        compiler_params=pltpu.CompilerParams(use_tc_tiling_on_sc=False),
        out_type=jax.ShapeDtypeStruct((S * NP * HEADS, HD), jnp.float32),
        scratch_types=[
            pltpu.VMEM((SBU * CPL,), jnp.int32),
            pltpu.VMEM((SBU * CPL,), jnp.float32),
            pltpu.VMEM((CPL, HD), jnp.float32),
            pltpu.VMEM((CPL, HD), jnp.float32),
            pltpu.VMEM((SBU * HEADS, HD), jnp.float32),
            pltpu.SemaphoreType.DMA,
            pltpu.SemaphoreType.DMA,
        ],
    )
    def run(table_hbm, idx_hbm, wgt_hbm, o_hbm, idxb, wgtb, rows0, rows1,
            outb, sem0, sem1):
        wid = lax.axis_index("s") * NC + lax.axis_index("c")
        base_u = wid * UPW

        def start(i, rbuf, sem_):
            pltpu.make_async_copy(
                table_hbm.at[idxb.at[pl.ds(i * CPL, CPL)]], rbuf, sem_
            ).start()

        def wait(rbuf, sem_):
            pltpu.make_async_copy(
                table_hbm.at[idxb.at[pl.ds(0, CPL)]], rbuf, sem_
            ).wait()

        def compute(i, rows):
            accs = [[jnp.zeros((16,), jnp.float32),
                     jnp.zeros((16,), jnp.float32)] for _ in range(HEADS)]
            for c in range(4):
                wv0 = wgtb[pl.ds(i * CPL + c * LANES, 16)]
                wv1 = wgtb[pl.ds(i * CPL + c * LANES + 16, 16)]
                for h in range(HEADS):
                    wv = wv0 if h < 2 else wv1
                    for p in range(POINTS):
                        w = wv[(h % 2) * POINTS + p]
                        r = c * LANES + h * POINTS + p
                        accs[h][0] = accs[h][0] + rows[r, pl.ds(0, 16)] * w
                        accs[h][1] = accs[h][1] + rows[r, pl.ds(16, 16)] * w
            for h in range(HEADS):
                outb[i * HEADS + h, pl.ds(0, 16)] = accs[h][0]
                outb[i * HEADS + h, pl.ds(16, 16)] = accs[h][1]

        def sb_body(sb, carry):
            u0 = base_u + sb * SBU
            pltpu.sync_copy(idx_hbm.at[pl.ds(u0 * CPL, SBU * CPL)], idxb)
            pltpu.sync_copy(wgt_hbm.at[pl.ds(u0 * CPL, SBU * CPL)], wgtb)
            start(0, rows0, sem0)

            def pair_body(j, c2):
                wait(rows0, sem0)
                start(2 * j + 1, rows1, sem1)
                compute(2 * j, rows0)
                wait(rows1, sem1)

                @pl.when(j < SBU // 2 - 1)
                def _():
                    start(2 * j + 2, rows0, sem0)

                compute(2 * j + 1, rows1)
                return c2

            lax.fori_loop(0, SBU // 2, pair_body, 0)
            pltpu.sync_copy(outb, o_hbm.at[pl.ds(u0 * HEADS, SBU * HEADS)])
            return carry

        lax.fori_loop(0, NSB, sb_body, 0)

    return run(table, idxf, wgtf)


def _k3_body(o_ref, m_ref, q_ref, wo_ref, bo_ref, out_ref):
    act = (jnp.sum(m_ref[...], axis=2) > 0.0).astype(jnp.float32)
    o = o_ref[...]
    acc = jnp.sum(o * act[:, :, None], axis=0)
    cnt = jnp.maximum(jnp.sum(act, axis=0), 1.0)
    slots = acc / cnt[:, None]
    out_ref[...] = (
        jnp.dot(slots, wo_ref[...], preferred_element_type=jnp.float32, precision=lax.Precision.HIGHEST)
        + bo_ref[0][None, :]
        + q_ref[...]
    )


def _finish(o, maskf, qpad, Wo, bo):
    nb = NP // 256
    return pl.pallas_call(
        _k3_body,
        grid=(nb,),
        in_specs=[
            pl.BlockSpec((S, 256, C), lambda n: (0, n, 0)),
            pl.BlockSpec((S, 256, D), lambda n: (0, n, 0)),
            pl.BlockSpec((256, C), lambda n: (n, 0)),
            pl.BlockSpec((C, C), lambda n: (0, 0)),
            pl.BlockSpec((1, C), lambda n: (0, 0)),
        ],
        out_specs=pl.BlockSpec((256, C), lambda n: (n, 0)),
        out_shape=jax.ShapeDtypeStruct((NP, C), jnp.float32),
    )(o, maskf, qpad, Wo, bo.reshape(1, C))


def kernel(query, key, value, reference_points_cam, spatial_shapes, bev_mask,
           Wv, bv, Ws, bs_, Wa, ba, Wo, bo):
    del key, spatial_shapes
    vflat4 = value[:, :, 0, :].reshape(S * M // 4, 4 * C)
    qpad = jnp.pad(query[0], ((0, NP - N), (0, 0)))
    rp8 = jnp.pad(reference_points_cam[:, 0].reshape(S, N, 2 * D),
                  ((0, 0), (0, NP - N), (0, 0)))
    maskf = jnp.pad(bev_mask[:, 0].astype(jnp.float32),
                    ((0, 0), (0, NP - N), (0, 0)))

    table = _value_table(vflat4, Wv, bv).reshape(S * HEADS * M, HD)
    idx, wgt = _corner_data(qpad, rp8, Ws, bs_, Wa, ba)
    o = _sc_sample(table, idx.reshape(S * NP * CPL), wgt.reshape(S * NP * CPL))
    o = o.reshape(S, NP, C)
    out = _finish(o, maskf, qpad, Wo, bo)
    return out[None, :N, :]


# speedup vs baseline: 1716.2222x; 1.0041x over previous
import functools

import jax
import jax.numpy as jnp
from jax import lax
from jax.experimental import pallas as pl
from jax.experimental.pallas import tpu as pltpu
from jax.experimental.pallas import tpu_sc as plsc

B, N, C = 1, 2500, 128
S, M = 6, 4096
D = 4
H, W = 64, 64
HEADS, POINTS = 4, 8
HD = C // HEADS
NP = 2560
LANES = HEADS * POINTS
CPL = 4 * LANES

NC, NS = 2, 16
NW = NC * NS
UPW = (S * NP) // NW
SBU = 32
NSB = UPW // SBU


def _k1_body(v_ref, wv_ref, bv_ref, out_ref):
    out_ref[...] = (
        jnp.dot(v_ref[...], wv_ref[0], preferred_element_type=jnp.float32)
        + bv_ref[0, 0][None, :]
    )


def _value_table(vflat4, Wv, bv):
    mb = 8
    wv_r = Wv.reshape(C, HEADS, HD)
    wbig = (jnp.eye(4, dtype=jnp.float32)[:, :, None, None, None]
            * wv_r[None, None])
    wbig = wbig.transpose(3, 0, 2, 1, 4).reshape(HEADS, 4 * C, C)
    bvt = jnp.tile(bv.reshape(HEADS, 1, HD), (1, 1, 4))
    return pl.pallas_call(
        _k1_body,
        grid=(HEADS, S, mb),
        in_specs=[
            pl.BlockSpec((128, 4 * C), lambda h, s, m: (s * mb + m, 0)),
            pl.BlockSpec((1, 4 * C, C), lambda h, s, m: (h, 0, 0)),
            pl.BlockSpec((1, 1, C), lambda h, s, m: (h, 0, 0)),
        ],
        out_specs=pl.BlockSpec((128, C), lambda h, s, m: ((s * HEADS + h) * mb + m, 0)),
        out_shape=jax.ShapeDtypeStruct((S * HEADS * M // 4, C), jnp.float32),
    )(vflat4, wbig, bvt)


def _iota2(shape, dim):
    return lax.broadcasted_iota(jnp.int32, shape, dim)


def _k2_body(q_ref, rp_ref, ws_ref, bs_ref, wa_ref, ba_ref, idx_ref, wgt_ref):
    q = q_ref[...]
    off = jnp.dot(q, ws_ref[...], preferred_element_type=jnp.float32, precision=lax.Precision.HIGHEST) + bs_ref[0][None, :]
    a = jnp.dot(q, wa_ref[...], preferred_element_type=jnp.float32, precision=lax.Precision.HIGHEST) + ba_ref[0][None, :]

    a = a - jnp.max(a, axis=-1, keepdims=True)
    e = jnp.exp(a)
    seg = (_iota2((LANES, LANES), 0) // POINTS == _iota2((LANES, LANES), 1) // POINTS)
    attnw = e / jnp.dot(e, seg.astype(jnp.float32), preferred_element_type=jnp.float32, precision=lax.Precision.HIGHEST)

    j64 = _iota2((2 * LANES, LANES), 0)
    k64 = _iota2((2 * LANES, LANES), 1)
    ox = jnp.dot(off, (j64 == 2 * k64).astype(jnp.float32), preferred_element_type=jnp.float32, precision=lax.Precision.HIGHEST)
    oy = jnp.dot(off, (j64 == 2 * k64 + 1).astype(jnp.float32), preferred_element_type=jnp.float32, precision=lax.Precision.HIGHEST)

    j8 = _iota2((2 * D, LANES), 0)
    d8 = (_iota2((2 * D, LANES), 1) % POINTS) % D
    selx = (j8 == 2 * d8).astype(jnp.float32)
    sely = (j8 == 2 * d8 + 1).astype(jnp.float32)
    hlane = _iota2((256, LANES), 1) // POINTS

    for s in range(S):
        rp8 = rp_ref[s]
        rpx = jnp.dot(rp8, selx, preferred_element_type=jnp.float32, precision=lax.Precision.HIGHEST)
        rpy = jnp.dot(rp8, sely, preferred_element_type=jnp.float32, precision=lax.Precision.HIGHEST)

        x = rpx * W + ox - 0.5
        y = rpy * H + oy - 0.5
        x0 = jnp.floor(x)
        y0 = jnp.floor(y)
        fx = x - x0
        fy = y - y0

        base = (s * HEADS + hlane) * M

        idxs = []
        wgts = []
        for cx, wx in ((x0, 1.0 - fx), (x0 + 1.0, fx)):
            for cy, wy in ((y0, 1.0 - fy), (y0 + 1.0, fy)):
                valid = ((cx >= 0.0) & (cx <= W - 1.0) & (cy >= 0.0) & (cy <= H - 1.0))
                wgts.append(wx * wy * attnw * valid.astype(jnp.float32))
                xi = jnp.clip(cx, 0.0, W - 1.0).astype(jnp.int32)
                yi = jnp.clip(cy, 0.0, H - 1.0).astype(jnp.int32)
                idxs.append(base + yi * W + xi)
        idx_ref[s] = jnp.concatenate(idxs, axis=-1)
        wgt_ref[s] = jnp.concatenate(wgts, axis=-1)


def _corner_data(qpad, rp8, Ws, bs_, Wa, ba):
    nb = NP // 256
    return pl.pallas_call(
        _k2_body,
        grid=(nb,),
        in_specs=[
            pl.BlockSpec((256, C), lambda n: (n, 0)),
            pl.BlockSpec((S, 256, 2 * D), lambda n: (0, n, 0)),
            pl.BlockSpec((C, 2 * LANES), lambda n: (0, 0)),
            pl.BlockSpec((1, 2 * LANES), lambda n: (0, 0)),
            pl.BlockSpec((C, LANES), lambda n: (0, 0)),
            pl.BlockSpec((1, LANES), lambda n: (0, 0)),
        ],
        out_specs=[
            pl.BlockSpec((S, 256, CPL), lambda n: (0, n, 0)),
            pl.BlockSpec((S, 256, CPL), lambda n: (0, n, 0)),
        ],
        out_shape=[
            jax.ShapeDtypeStruct((S, NP, CPL), jnp.int32),
            jax.ShapeDtypeStruct((S, NP, CPL), jnp.float32),
        ],
    )(qpad, rp8, Ws, bs_.reshape(1, 2 * LANES), Wa, ba.reshape(1, LANES))


def _sc_sample(table, idxf, wgtf):
    mesh = plsc.VectorSubcoreMesh(core_axis_name="c", subcore_axis_name="s")

    @functools.partial(
        pl.kernel,
        mesh=mesh,
        compiler_params=pltpu.CompilerParams(use_tc_tiling_on_sc=False),
        out_type=jax.ShapeDtypeStruct((S * NP * HEADS, HD), jnp.float32),
        scratch_types=[
            pltpu.VMEM((SBU * CPL,), jnp.int32),
            pltpu.VMEM((SBU * CPL,), jnp.float32),
            pltpu.VMEM((CPL, HD), jnp.float32),
            pltpu.VMEM((CPL, HD), jnp.float32),
            pltpu.VMEM((SBU * HEADS, HD), jnp.float32),
            pltpu.SemaphoreType.DMA,
            pltpu.SemaphoreType.DMA,
        ],
    )
    def run(table_hbm, idx_hbm, wgt_hbm, o_hbm, idxb, wgtb, rows0, rows1,
            outb, sem0, sem1):
        wid = lax.axis_index("s") * NC + lax.axis_index("c")
        base_u = wid * UPW

        def start(i, rbuf, sem_):
            pltpu.make_async_copy(
                table_hbm.at[idxb.at[pl.ds(i * CPL, CPL)]], rbuf, sem_
            ).start()

        def wait(rbuf, sem_):
            pltpu.make_async_copy(
                table_hbm.at[idxb.at[pl.ds(0, CPL)]], rbuf, sem_
            ).wait()

        def compute(i, rows):
            accs = [[jnp.zeros((16,), jnp.float32),
                     jnp.zeros((16,), jnp.float32)] for _ in range(HEADS)]
            for c in range(4):
                wv0 = wgtb[pl.ds(i * CPL + c * LANES, 16)]
                wv1 = wgtb[pl.ds(i * CPL + c * LANES + 16, 16)]
                for h in range(HEADS):
                    wv = wv0 if h < 2 else wv1
                    for p in range(POINTS):
                        w = wv[(h % 2) * POINTS + p]
                        r = c * LANES + h * POINTS + p
                        accs[h][0] = accs[h][0] + rows[r, pl.ds(0, 16)] * w
                        accs[h][1] = accs[h][1] + rows[r, pl.ds(16, 16)] * w
            for h in range(HEADS):
                outb[i * HEADS + h, pl.ds(0, 16)] = accs[h][0]
                outb[i * HEADS + h, pl.ds(16, 16)] = accs[h][1]

        def sb_body(sb, carry):
            u0 = base_u + sb * SBU
            pltpu.sync_copy(idx_hbm.at[pl.ds(u0 * CPL, SBU * CPL)], idxb)
            pltpu.sync_copy(wgt_hbm.at[pl.ds(u0 * CPL, SBU * CPL)], wgtb)
            start(0, rows0, sem0)

            def pair_body(j, c2):
                wait(rows0, sem0)
                start(2 * j + 1, rows1, sem1)
                compute(2 * j, rows0)
                wait(rows1, sem1)

                @pl.when(j < SBU // 2 - 1)
                def _():
                    start(2 * j + 2, rows0, sem0)

                compute(2 * j + 1, rows1)
                return c2

            lax.fori_loop(0, SBU // 2, pair_body, 0)
            pltpu.sync_copy(outb, o_hbm.at[pl.ds(u0 * HEADS, SBU * HEADS)])
            return carry

        lax.fori_loop(0, NSB, sb_body, 0)

    return run(table, idxf, wgtf)


def _k3_body(o_ref, m_ref, q_ref, wo_ref, bo_ref, out_ref):
    act = (jnp.sum(m_ref[...], axis=2) > 0.0).astype(jnp.float32)
    o = o_ref[...]
    acc = jnp.sum(o * act[:, :, None], axis=0)
    cnt = jnp.maximum(jnp.sum(act, axis=0), 1.0)
    slots = acc / cnt[:, None]
    out_ref[...] = (
        jnp.dot(slots, wo_ref[...], preferred_element_type=jnp.float32, precision=lax.Precision.HIGHEST)
        + bo_ref[0][None, :]
        + q_ref[...]
    )


def _finish(o, maskf, qpad, Wo, bo):
    nb = NP // 256
    return pl.pallas_call(
        _k3_body,
        grid=(nb,),
        in_specs=[
            pl.BlockSpec((S, 256, C), lambda n: (0, n, 0)),
            pl.BlockSpec((S, 256, D), lambda n: (0, n, 0)),
            pl.BlockSpec((256, C), lambda n: (n, 0)),
            pl.BlockSpec((C, C), lambda n: (0, 0)),
            pl.BlockSpec((1, C), lambda n: (0, 0)),
        ],
        out_specs=pl.BlockSpec((256, C), lambda n: (n, 0)),
        out_shape=jax.ShapeDtypeStruct((NP, C), jnp.float32),
    )(o, maskf, qpad, Wo, bo.reshape(1, C))


def kernel(query, key, value, reference_points_cam, spatial_shapes, bev_mask,
           Wv, bv, Ws, bs_, Wa, ba, Wo, bo):
    del key, spatial_shapes
    vflat4 = value.reshape(S * M // 4, 4 * C)
    qpad = jnp.pad(query[0], ((0, NP - N), (0, 0)))
    rp8 = jnp.pad(reference_points_cam[:, 0].reshape(S, N, 2 * D),
                  ((0, 0), (0, NP - N), (0, 0)))
    maskf = jnp.pad(bev_mask[:, 0].astype(jnp.float32),
                    ((0, 0), (0, NP - N), (0, 0)))

    table = _value_table(vflat4, Wv, bv).reshape(S * HEADS * M, HD)
    idx, wgt = _corner_data(qpad, rp8, Ws, bs_, Wa, ba)
    o = _sc_sample(table, idx.reshape(S * NP * CPL), wgt.reshape(S * NP * CPL))
    o = o.reshape(S, NP, C)
    out = _finish(o, maskf, qpad, Wo, bo)
    return out[None, :N, :]


# speedup vs baseline: 2411.2431x; 1.4050x over previous
import functools

import jax
import jax.numpy as jnp
from jax import lax
from jax.experimental import pallas as pl
from jax.experimental.pallas import tpu as pltpu
from jax.experimental.pallas import tpu_sc as plsc

B, N, C = 1, 2500, 128
S, M = 6, 4096
D = 4
H, W = 64, 64
HEADS, POINTS = 4, 8
HD = C // HEADS
NP = 2560
LANES = HEADS * POINTS
CPL = 4 * LANES

NC, NS = 2, 16
NW = NC * NS
UPW = (S * NP) // NW
SBU = 32
NSB = UPW // SBU


def _k1_body(v_ref, wv_ref, bv_ref, out_ref):
    out_ref[...] = (
        jnp.dot(v_ref[...], wv_ref[0], preferred_element_type=jnp.float32)
        + bv_ref[0, 0][None, :]
    )


def _value_table(vflat4, Wv, bv):
    mb = 8
    wv_r = Wv.reshape(C, HEADS, HD)
    wbig = (jnp.eye(4, dtype=jnp.float32)[:, :, None, None, None]
            * wv_r[None, None])
    wbig = wbig.transpose(3, 0, 2, 1, 4).reshape(HEADS, 4 * C, C)
    bvt = jnp.tile(bv.reshape(HEADS, 1, HD), (1, 1, 4))
    return pl.pallas_call(
        _k1_body,
        grid=(HEADS, S, mb),
        in_specs=[
            pl.BlockSpec((128, 4 * C), lambda h, s, m: (s * mb + m, 0)),
            pl.BlockSpec((1, 4 * C, C), lambda h, s, m: (h, 0, 0)),
            pl.BlockSpec((1, 1, C), lambda h, s, m: (h, 0, 0)),
        ],
        out_specs=pl.BlockSpec((128, C), lambda h, s, m: ((s * HEADS + h) * mb + m, 0)),
        out_shape=jax.ShapeDtypeStruct((S * HEADS * M // 4, C), jnp.float32),
    )(vflat4, wbig, bvt)


def _iota2(shape, dim):
    return lax.broadcasted_iota(jnp.int32, shape, dim)


def _k2_body(q_ref, rp_ref, ws_ref, bs_ref, wa_ref, ba_ref, idx_ref, wgt_ref):
    q = q_ref[...]
    off = jnp.dot(q, ws_ref[...], preferred_element_type=jnp.float32, precision=lax.Precision.HIGHEST) + bs_ref[0][None, :]
    a = jnp.dot(q, wa_ref[...], preferred_element_type=jnp.float32, precision=lax.Precision.HIGHEST) + ba_ref[0][None, :]

    a = a - jnp.max(a, axis=-1, keepdims=True)
    e = jnp.exp(a)
    seg = (_iota2((LANES, LANES), 0) // POINTS == _iota2((LANES, LANES), 1) // POINTS)
    attnw = e / jnp.dot(e, seg.astype(jnp.float32), preferred_element_type=jnp.float32, precision=lax.Precision.HIGHEST)

    j64 = _iota2((2 * LANES, LANES), 0)
    k64 = _iota2((2 * LANES, LANES), 1)
    ox = jnp.dot(off, (j64 == 2 * k64).astype(jnp.float32), preferred_element_type=jnp.float32, precision=lax.Precision.HIGHEST)
    oy = jnp.dot(off, (j64 == 2 * k64 + 1).astype(jnp.float32), preferred_element_type=jnp.float32, precision=lax.Precision.HIGHEST)

    j8 = _iota2((2 * D, LANES), 0)
    d8 = (_iota2((2 * D, LANES), 1) % POINTS) % D
    selx = (j8 == 2 * d8).astype(jnp.float32)
    sely = (j8 == 2 * d8 + 1).astype(jnp.float32)
    hlane = _iota2((256, LANES), 1) // POINTS

    for s in range(S):
        rp8 = rp_ref[s]
        rpx = jnp.dot(rp8, selx, preferred_element_type=jnp.float32, precision=lax.Precision.HIGHEST)
        rpy = jnp.dot(rp8, sely, preferred_element_type=jnp.float32, precision=lax.Precision.HIGHEST)

        x = rpx * W + ox - 0.5
        y = rpy * H + oy - 0.5
        x0 = jnp.floor(x)
        y0 = jnp.floor(y)
        fx = x - x0
        fy = y - y0

        base = (s * HEADS + hlane) * M

        idxs = []
        wgts = []
        for cx, wx in ((x0, 1.0 - fx), (x0 + 1.0, fx)):
            for cy, wy in ((y0, 1.0 - fy), (y0 + 1.0, fy)):
                valid = ((cx >= 0.0) & (cx <= W - 1.0) & (cy >= 0.0) & (cy <= H - 1.0))
                wgts.append(wx * wy * attnw * valid.astype(jnp.float32))
                xi = jnp.clip(cx, 0.0, W - 1.0).astype(jnp.int32)
                yi = jnp.clip(cy, 0.0, H - 1.0).astype(jnp.int32)
                idxs.append(base + yi * W + xi)
        idx_ref[s] = jnp.concatenate(idxs, axis=-1)
        wgt_ref[s] = jnp.concatenate(wgts, axis=-1)


def _corner_data(qpad, rp8, Ws, bs_, Wa, ba):
    nb = NP // 256
    return pl.pallas_call(
        _k2_body,
        grid=(nb,),
        in_specs=[
            pl.BlockSpec((256, C), lambda n: (n, 0)),
            pl.BlockSpec((S, 256, 2 * D), lambda n: (0, n, 0)),
            pl.BlockSpec((C, 2 * LANES), lambda n: (0, 0)),
            pl.BlockSpec((1, 2 * LANES), lambda n: (0, 0)),
            pl.BlockSpec((C, LANES), lambda n: (0, 0)),
            pl.BlockSpec((1, LANES), lambda n: (0, 0)),
        ],
        out_specs=[
            pl.BlockSpec((S, 256, CPL), lambda n: (0, n, 0)),
            pl.BlockSpec((S, 256, CPL), lambda n: (0, n, 0)),
        ],
        out_shape=[
            jax.ShapeDtypeStruct((S, NP, CPL), jnp.int32),
            jax.ShapeDtypeStruct((S, NP, CPL), jnp.float32),
        ],
    )(qpad, rp8, Ws, bs_.reshape(1, 2 * LANES), Wa, ba.reshape(1, LANES))


def _sc_sample(table, idxf, wgtf):
    mesh = plsc.VectorSubcoreMesh(core_axis_name="c", subcore_axis_name="s")

    @functools.partial(
        pl.kernel,
        mesh=mesh,
        compiler_params=pltpu.CompilerParams(use_tc_tiling_on_sc=False),
        out_type=jax.ShapeDtypeStruct((S * NP * HEADS, HD), jnp.float32),
        scratch_types=[
            pltpu.VMEM((SBU * CPL,), jnp.int32),
            pltpu.VMEM((SBU * CPL,), jnp.float32),
            pltpu.VMEM((CPL, HD), jnp.float32),
            pltpu.VMEM((CPL, HD), jnp.float32),
            pltpu.VMEM((CPL, HD), jnp.float32),
            pltpu.VMEM((CPL, HD), jnp.float32),
            pltpu.VMEM((SBU * HEADS, HD), jnp.float32),
            pltpu.SemaphoreType.DMA,
            pltpu.SemaphoreType.DMA,
            pltpu.SemaphoreType.DMA,
            pltpu.SemaphoreType.DMA,
        ],
    )
    def run(table_hbm, idx_hbm, wgt_hbm, o_hbm, idxb, wgtb, rows0, rows1,
            rows2, rows3, outb, sem0, sem1, sem2, sem3):
        wid = lax.axis_index("s") * NC + lax.axis_index("c")
        base_u = wid * UPW

        def start(i, rbuf, sem_):
            pltpu.make_async_copy(
                table_hbm.at[idxb.at[pl.ds(i * CPL, CPL)]], rbuf, sem_
            ).start()

        def wait(rbuf, sem_):
            pltpu.make_async_copy(
                table_hbm.at[idxb.at[pl.ds(0, CPL)]], rbuf, sem_
            ).wait()

        def compute(i, rows):
            accs = [[jnp.zeros((16,), jnp.float32),
                     jnp.zeros((16,), jnp.float32)] for _ in range(HEADS)]
            for c in range(4):
                wv0 = wgtb[pl.ds(i * CPL + c * LANES, 16)]
                wv1 = wgtb[pl.ds(i * CPL + c * LANES + 16, 16)]
                for h in range(HEADS):
                    wv = wv0 if h < 2 else wv1
                    for p in range(POINTS):
                        w = wv[(h % 2) * POINTS + p]
                        r = c * LANES + h * POINTS + p
                        accs[h][0] = accs[h][0] + rows[r, pl.ds(0, 16)] * w
                        accs[h][1] = accs[h][1] + rows[r, pl.ds(16, 16)] * w
            for h in range(HEADS):
                outb[i * HEADS + h, pl.ds(0, 16)] = accs[h][0]
                outb[i * HEADS + h, pl.ds(16, 16)] = accs[h][1]

        ring = ((rows0, sem0), (rows1, sem1), (rows2, sem2), (rows3, sem3))

        def sb_body(sb, carry):
            u0 = base_u + sb * SBU
            pltpu.sync_copy(idx_hbm.at[pl.ds(u0 * CPL, SBU * CPL)], idxb)
            pltpu.sync_copy(wgt_hbm.at[pl.ds(u0 * CPL, SBU * CPL)], wgtb)
            for i in range(3):
                start(i, *ring[i])

            def quad_body(j, c2):
                for i in range(4):
                    k = 4 * j + i
                    wait(*ring[i])
                    nxt = k + 3

                    @pl.when(nxt < SBU)
                    def _():
                        start(nxt, *ring[(i + 3) % 4])

                    compute(k, ring[i][0])
                return c2

            lax.fori_loop(0, SBU // 4, quad_body, 0)
            pltpu.sync_copy(outb, o_hbm.at[pl.ds(u0 * HEADS, SBU * HEADS)])
            return carry

        lax.fori_loop(0, NSB, sb_body, 0)

    return run(table, idxf, wgtf)


def _k3_body(o_ref, m_ref, q_ref, wo_ref, bo_ref, out_ref):
    act = (jnp.sum(m_ref[...], axis=2) > 0.0).astype(jnp.float32)
    o = o_ref[...]
    acc = jnp.sum(o * act[:, :, None], axis=0)
    cnt = jnp.maximum(jnp.sum(act, axis=0), 1.0)
    slots = acc / cnt[:, None]
    out_ref[...] = (
        jnp.dot(slots, wo_ref[...], preferred_element_type=jnp.float32, precision=lax.Precision.HIGHEST)
        + bo_ref[0][None, :]
        + q_ref[...]
    )


def _finish(o, maskf, qpad, Wo, bo):
    nb = NP // 256
    return pl.pallas_call(
        _k3_body,
        grid=(nb,),
        in_specs=[
            pl.BlockSpec((S, 256, C), lambda n: (0, n, 0)),
            pl.BlockSpec((S, 256, D), lambda n: (0, n, 0)),
            pl.BlockSpec((256, C), lambda n: (n, 0)),
            pl.BlockSpec((C, C), lambda n: (0, 0)),
            pl.BlockSpec((1, C), lambda n: (0, 0)),
        ],
        out_specs=pl.BlockSpec((256, C), lambda n: (n, 0)),
        out_shape=jax.ShapeDtypeStruct((NP, C), jnp.float32),
    )(o, maskf, qpad, Wo, bo.reshape(1, C))


def kernel(query, key, value, reference_points_cam, spatial_shapes, bev_mask,
           Wv, bv, Ws, bs_, Wa, ba, Wo, bo):
    del key, spatial_shapes
    vflat4 = value.reshape(S * M // 4, 4 * C)
    qpad = jnp.pad(query[0], ((0, NP - N), (0, 0)))
    rp8 = jnp.pad(reference_points_cam[:, 0].reshape(S, N, 2 * D),
                  ((0, 0), (0, NP - N), (0, 0)))
    maskf = jnp.pad(bev_mask[:, 0].astype(jnp.float32),
                    ((0, 0), (0, NP - N), (0, 0)))

    table = _value_table(vflat4, Wv, bv).reshape(S * HEADS * M, HD)
    idx, wgt = _corner_data(qpad, rp8, Ws, bs_, Wa, ba)
    o = _sc_sample(table, idx.reshape(S * NP * CPL), wgt.reshape(S * NP * CPL))
    o = o.reshape(S, NP, C)
    out = _finish(o, maskf, qpad, Wo, bo)
    return out[None, :N, :]
